# baseline (speedup 1.0000x reference)
import functools

import jax
import jax.numpy as jnp
from jax import lax
from jax.experimental import pallas as pl
from jax.experimental.pallas import tpu as pltpu
from jax.experimental.pallas import tpu_sc as plsc

EMB = 128
NTAB = 512
L = 16
NW = 32
NSUB = 16
C = 80
NBUF = 5
EA_BLK = 2000
RPT = NTAB // NSUB


def _make_sc_kernel(E):
    per_tile = E // NW
    n_chunks = per_tile // C
    n_ea = per_tile // EA_BLK
    mesh = plsc.VectorSubcoreMesh(core_axis_name="c", subcore_axis_name="s")

    @functools.partial(
        pl.kernel,
        mesh=mesh,
        out_type=jax.ShapeDtypeStruct((E, EMB), jnp.float32),
        scratch_types=[
            pltpu.VMEM((per_tile,), jnp.int32),
            pltpu.VMEM((EA_BLK,), jnp.int32),
            pltpu.VMEM((EA_BLK,), jnp.int32),
            pltpu.VMEM((EA_BLK,), jnp.int32),
            pltpu.VMEM((NSUB * EMB,), jnp.float32),
            pltpu.VMEM((NSUB * EMB,), jnp.float32),
            pltpu.VMEM((NSUB * EMB,), jnp.float32),
            pltpu.VMEM((RPT, EMB), jnp.float32),
            pltpu.VMEM_SHARED((NTAB, EMB), jnp.float32),
        ] + [pltpu.VMEM((C, EMB), jnp.float32) for _ in range(NBUF)]
        + [pltpu.SemaphoreType.DMA for _ in range(2 * NBUF)],
    )
    def sc_kernel(ea_hbm, w0_hbm, w1_hbm, w2_hbm, out_hbm,
                  idx_v, e0_v, e1_v, e2_v, w0_v, w1_v, w2_v, trows,
                  table_sh, *bufs_and_sems):
        rows = bufs_and_sems[:NBUF]
        sg = bufs_and_sems[NBUF:2 * NBUF]
        ss = bufs_and_sems[2 * NBUF:]
        cid = lax.axis_index("c")
        sid = lax.axis_index("s")
        wid = sid * 2 + cid
        base = wid * per_tile

        pltpu.sync_copy(w0_hbm, w0_v)
        pltpu.sync_copy(w1_hbm, w1_v)
        pltpu.sync_copy(w2_hbm, w2_v)
        c2 = (sid * RPT) >> 6
        for blk in range(RPT // 8):
            c1 = ((sid * (RPT // 8) + blk) & 7)
            for r0 in range(8):
                r = blk * 8 + r0
                for j in range(EMB // L):
                    s = pl.ds(j * L, L)
                    trows[r, s] = (
                        w0_v[pl.ds(r0 * EMB + j * L, L)]
                        + w1_v[pl.ds(c1 * EMB + j * L, L)]
                        + w2_v[pl.ds(c2 * EMB + j * L, L)]
                    )
        pltpu.sync_copy(trows, table_sh.at[pl.ds(sid * RPT, RPT)])

        def ea_round(rr, carry):
            off = base + rr * EA_BLK
            pltpu.sync_copy(ea_hbm.at[pl.ds(off, EA_BLK)], e0_v)
            pltpu.sync_copy(ea_hbm.at[pl.ds(E + off, EA_BLK)], e1_v)
            pltpu.sync_copy(ea_hbm.at[pl.ds(2 * E + off, EA_BLK)], e2_v)
            for g in range(EA_BLK // L):
                s = pl.ds(g * L, L)
                idx_v[pl.ds(rr * EA_BLK + g * L, L)] = (
                    e0_v[s] + (e1_v[s] << 3) + (e2_v[s] << 6)
                )
            return carry

        lax.fori_loop(0, n_ea, ea_round, 0)
        plsc.subcore_barrier()

        def g_copy(c, b):
            return pltpu.make_async_copy(
                table_sh.at[idx_v.at[pl.ds(c * C, C)]], rows[b], sg[b])

        def s_copy(c, b):
            return pltpu.make_async_copy(
                rows[b], out_hbm.at[pl.ds(base + c * C, C)], ss[b])

        n_groups = n_chunks // NBUF
        for b in range(NBUF):
            g_copy(b, b).start()

        def pipe_group(i, carry):
            c0 = i * NBUF
            for b in range(NBUF):
                g_copy(c0 + b, b).wait()
                s_copy(c0 + b, b).start()
            for b in range(NBUF):
                s_copy(c0 + b, b).wait()
                g_copy(c0 + NBUF + b, b).start()
            return carry

        lax.fori_loop(0, n_groups - 1, pipe_group, 0)

        cl = (n_groups - 1) * NBUF
        for b in range(NBUF):
            g_copy(cl + b, b).wait()
            s_copy(cl + b, b).start()
        for b in range(NBUF):
            s_copy(cl + b, b).wait()

    return sc_kernel


def kernel(edge_attr, W0, W1, W2):
    E = edge_attr.shape[0]
    ea = edge_attr.astype(jnp.int32).T.reshape(-1)
    wf = [jnp.pad(w, ((0, NSUB - w.shape[0]), (0, 0))).reshape(-1)
          for w in (W0, W1, W2)]
    return _make_sc_kernel(E)(ea, *wf)


# speedup vs baseline: 1.1439x; 1.1439x over previous
import functools

import jax
import jax.numpy as jnp
from jax import lax
from jax.experimental import pallas as pl
from jax.experimental.pallas import tpu as pltpu
from jax.experimental.pallas import tpu_sc as plsc

EMB = 128
NTAB = 512
L = 16
NW = 32
NSUB = 16
C = 80
NBUF = 5
RPT = NTAB // NSUB


def _make_sc_kernel(E):
    per_tile = E // NW
    n_chunks = per_tile // C
    win = NBUF * C
    mesh = plsc.VectorSubcoreMesh(core_axis_name="c", subcore_axis_name="s")

    @functools.partial(
        pl.kernel,
        mesh=mesh,
        out_type=jax.ShapeDtypeStruct((E, EMB), jnp.float32),
        scratch_types=[
            pltpu.VMEM((per_tile,), jnp.int32),
            pltpu.VMEM((per_tile,), jnp.int32),
            pltpu.VMEM((per_tile,), jnp.int32),
            pltpu.VMEM((per_tile,), jnp.int32),
            pltpu.VMEM((NSUB * EMB,), jnp.float32),
            pltpu.VMEM((NSUB * EMB,), jnp.float32),
            pltpu.VMEM((NSUB * EMB,), jnp.float32),
            pltpu.VMEM((RPT, EMB), jnp.float32),
            pltpu.VMEM_SHARED((NTAB, EMB), jnp.float32),
            pltpu.SemaphoreType.DMA,
            pltpu.SemaphoreType.DMA,
        ] + [pltpu.VMEM((C, EMB), jnp.float32) for _ in range(NBUF)]
        + [pltpu.SemaphoreType.DMA for _ in range(2 * NBUF)],
    )
    def sc_kernel(ea_hbm, w0_hbm, w1_hbm, w2_hbm, out_hbm,
                  idx_v, e0_v, e1_v, e2_v, w0_v, w1_v, w2_v, trows,
                  table_sh, semw, seme, *bufs_and_sems):
        rows = bufs_and_sems[:NBUF]
        sg = bufs_and_sems[NBUF:2 * NBUF]
        ss = bufs_and_sems[2 * NBUF:]
        cid = lax.axis_index("c")
        sid = lax.axis_index("s")
        wid = sid * 2 + cid
        base = wid * per_tile

        w_copies = [
            pltpu.make_async_copy(w0_hbm, w0_v, semw),
            pltpu.make_async_copy(w1_hbm, w1_v, semw),
            pltpu.make_async_copy(w2_hbm, w2_v, semw),
        ]
        ea_copies = [
            pltpu.make_async_copy(
                ea_hbm.at[pl.ds(a * E + base, per_tile)], ev, seme)
            for a, ev in ((0, e0_v), (1, e1_v), (2, e2_v))
        ]
        for cp in w_copies + ea_copies:
            cp.start()

        for cp in w_copies:
            cp.wait()
        c2 = (sid * RPT) >> 6
        for blk in range(RPT // 8):
            c1 = ((sid * (RPT // 8) + blk) & 7)
            for r0 in range(8):
                r = blk * 8 + r0
                for j in range(EMB // L):
                    s = pl.ds(j * L, L)
                    trows[r, s] = (
                        w0_v[pl.ds(r0 * EMB + j * L, L)]
                        + w1_v[pl.ds(c1 * EMB + j * L, L)]
                        + w2_v[pl.ds(c2 * EMB + j * L, L)]
                    )
        trows_copy = pltpu.make_async_copy(
            trows, table_sh.at[pl.ds(sid * RPT, RPT)], semw)
        trows_copy.start()

        def idx_window(w0_, n):
            for g in range(n // L):
                s = pl.ds(w0_ + g * L, L)
                idx_v[s] = e0_v[s] + (e1_v[s] << 3) + (e2_v[s] << 6)

        for cp in ea_copies:
            cp.wait()
        idx_window(0, win)
        trows_copy.wait()
        plsc.subcore_barrier()

        def g_copy(c, b):
            return pltpu.make_async_copy(
                table_sh.at[idx_v.at[pl.ds(c * C, C)]], rows[b], sg[b])

        def s_copy(c, b):
            return pltpu.make_async_copy(
                rows[b], out_hbm.at[pl.ds(base + c * C, C)], ss[b])

        n_groups = n_chunks // NBUF
        for b in range(NBUF):
            g_copy(b, b).start()

        def pipe_group(i, carry):
            c0 = i * NBUF
            idx_window((i + 1) * win, win)
            for b in range(NBUF):
                g_copy(c0 + b, b).wait()
                s_copy(c0 + b, b).start()
            for b in range(NBUF):
                s_copy(c0 + b, b).wait()
                g_copy(c0 + NBUF + b, b).start()
            return carry

        lax.fori_loop(0, n_groups - 1, pipe_group, 0)

        cl = (n_groups - 1) * NBUF
        for b in range(NBUF):
            g_copy(cl + b, b).wait()
            s_copy(cl + b, b).start()
        for b in range(NBUF):
            s_copy(cl + b, b).wait()

    return sc_kernel


def kernel(edge_attr, W0, W1, W2):
    E = edge_attr.shape[0]
    ea = edge_attr.astype(jnp.int32).T.reshape(-1)
    wf = [jnp.pad(w, ((0, NSUB - w.shape[0]), (0, 0))).reshape(-1)
          for w in (W0, W1, W2)]
    return _make_sc_kernel(E)(ea, *wf)


# speedup vs baseline: 1.1536x; 1.0085x over previous
import functools

import jax
import jax.numpy as jnp
from jax import lax
from jax.experimental import pallas as pl
from jax.experimental.pallas import tpu as pltpu
from jax.experimental.pallas import tpu_sc as plsc

EMB = 128
NTAB = 512
L = 16
NW = 32
NSUB = 16
C = 40
NBUF = 10
RPT = NTAB // NSUB


def _make_sc_kernel(E):
    per_tile = E // NW
    n_chunks = per_tile // C
    win = NBUF * C
    mesh = plsc.VectorSubcoreMesh(core_axis_name="c", subcore_axis_name="s")

    @functools.partial(
        pl.kernel,
        mesh=mesh,
        out_type=jax.ShapeDtypeStruct((E, EMB), jnp.float32),
        scratch_types=[
            pltpu.VMEM((per_tile,), jnp.int32),
            pltpu.VMEM((per_tile,), jnp.int32),
            pltpu.VMEM((per_tile,), jnp.int32),
            pltpu.VMEM((per_tile,), jnp.int32),
            pltpu.VMEM((NSUB * EMB,), jnp.float32),
            pltpu.VMEM((NSUB * EMB,), jnp.float32),
            pltpu.VMEM((NSUB * EMB,), jnp.float32),
            pltpu.VMEM((RPT, EMB), jnp.float32),
            pltpu.VMEM_SHARED((NTAB, EMB), jnp.float32),
            pltpu.SemaphoreType.DMA,
            pltpu.SemaphoreType.DMA,
        ] + [pltpu.VMEM((C, EMB), jnp.float32) for _ in range(NBUF)]
        + [pltpu.SemaphoreType.DMA for _ in range(2 * NBUF)],
    )
    def sc_kernel(ea_hbm, w0_hbm, w1_hbm, w2_hbm, out_hbm,
                  idx_v, e0_v, e1_v, e2_v, w0_v, w1_v, w2_v, trows,
                  table_sh, semw, seme, *bufs_and_sems):
        rows = bufs_and_sems[:NBUF]
        sg = bufs_and_sems[NBUF:2 * NBUF]
        ss = bufs_and_sems[2 * NBUF:]
        cid = lax.axis_index("c")
        sid = lax.axis_index("s")
        wid = sid * 2 + cid
        base = wid * per_tile

        w_copies = [
            pltpu.make_async_copy(w0_hbm, w0_v, semw),
            pltpu.make_async_copy(w1_hbm, w1_v, semw),
            pltpu.make_async_copy(w2_hbm, w2_v, semw),
        ]
        ea_copies = [
            pltpu.make_async_copy(
                ea_hbm.at[pl.ds(a * E + base, per_tile)], ev, seme)
            for a, ev in ((0, e0_v), (1, e1_v), (2, e2_v))
        ]
        for cp in w_copies + ea_copies:
            cp.start()

        for cp in w_copies:
            cp.wait()
        c2 = (sid * RPT) >> 6
        for blk in range(RPT // 8):
            c1 = ((sid * (RPT // 8) + blk) & 7)
            for r0 in range(8):
                r = blk * 8 + r0
                for j in range(EMB // L):
                    s = pl.ds(j * L, L)
                    trows[r, s] = (
                        w0_v[pl.ds(r0 * EMB + j * L, L)]
                        + w1_v[pl.ds(c1 * EMB + j * L, L)]
                        + w2_v[pl.ds(c2 * EMB + j * L, L)]
                    )
        trows_copy = pltpu.make_async_copy(
            trows, table_sh.at[pl.ds(sid * RPT, RPT)], semw)
        trows_copy.start()

        def idx_window(w0_, n):
            for g in range(n // L):
                s = pl.ds(w0_ + g * L, L)
                idx_v[s] = e0_v[s] + (e1_v[s] << 3) + (e2_v[s] << 6)

        for cp in ea_copies:
            cp.wait()
        idx_window(0, win)
        trows_copy.wait()
        plsc.subcore_barrier()

        def g_copy(c, b):
            return pltpu.make_async_copy(
                table_sh.at[idx_v.at[pl.ds(c * C, C)]], rows[b], sg[b])

        def s_copy(c, b):
            return pltpu.make_async_copy(
                rows[b], out_hbm.at[pl.ds(base + c * C, C)], ss[b])

        n_groups = n_chunks // NBUF
        for b in range(NBUF):
            g_copy(b, b).start()

        def pipe_group(i, carry):
            c0 = i * NBUF
            idx_window((i + 1) * win, win)
            for b in range(NBUF):
                g_copy(c0 + b, b).wait()
                s_copy(c0 + b, b).start()
            for b in range(NBUF):
                s_copy(c0 + b, b).wait()
                g_copy(c0 + NBUF + b, b).start()
            return carry

        lax.fori_loop(0, n_groups - 1, pipe_group, 0)

        cl = (n_groups - 1) * NBUF
        for b in range(NBUF):
            g_copy(cl + b, b).wait()
            s_copy(cl + b, b).start()
        for b in range(NBUF):
            s_copy(cl + b, b).wait()

    return sc_kernel


def kernel(edge_attr, W0, W1, W2):
    E = edge_attr.shape[0]
    ea = edge_attr.astype(jnp.int32).T.reshape(-1)
    wf = [jnp.pad(w, ((0, NSUB - w.shape[0]), (0, 0))).reshape(-1)
          for w in (W0, W1, W2)]
    return _make_sc_kernel(E)(ea, *wf)
